# baseline (speedup 1.0000x reference)
import functools

import jax
import jax.numpy as jnp
from jax import lax
from jax.experimental import pallas as pl
from jax.experimental.pallas import tpu as pltpu
from jax.experimental.pallas import tpu_sc as plsc

_NUM_CORES = 2
_NUM_SUBCORES = 16
_NUM_WORKERS = _NUM_CORES * _NUM_SUBCORES
_CHUNK = 256
_NBUF = 3
_LANES = 128
_BLK = 16384


def _pad_body(t_ref, eye_ref, o_ref):
    del eye_ref
    xt = t_ref[...].T
    o_ref[...] = jnp.concatenate(
        [xt, jnp.zeros((xt.shape[0], _LANES - xt.shape[1]), jnp.float32)], axis=1
    )


@functools.partial(jax.jit, static_argnames=("vocab", "embed"))
def _pad_tc(table_t, eyepad, *, vocab, embed):
    return pl.pallas_call(
        _pad_body,
        grid=(pl.cdiv(vocab, _BLK),),
        in_specs=[
            pl.BlockSpec((embed, _BLK), lambda i: (0, i)),
            pl.BlockSpec((embed, _LANES), lambda i: (0, 0)),
        ],
        out_specs=pl.BlockSpec((_BLK, _LANES), lambda i: (i, 0)),
        out_shape=jax.ShapeDtypeStruct((vocab, _LANES), jnp.float32),
    )(table_t, eyepad)


@functools.partial(jax.jit, static_argnames=("seq", "embed"))
def _gather_sc(idx, table_pad, *, seq, embed):
    mesh = plsc.VectorSubcoreMesh(core_axis_name="c", subcore_axis_name="s")
    e_blocks = embed // 8

    @functools.partial(
        pl.kernel,
        out_type=jax.ShapeDtypeStruct(
            (seq, e_blocks, _NUM_WORKERS, 8, _LANES), jnp.float32
        ),
        mesh=mesh,
        compiler_params=pltpu.CompilerParams(
            use_tc_tiling_on_sc=False, needs_layout_passes=False
        ),
        scratch_types=[
            pltpu.VMEM((seq, _LANES), jnp.int32),
            pltpu.VMEM((2, _LANES, _LANES), jnp.float32),
            pltpu.VMEM((2, e_blocks, 8, _LANES), jnp.float32),
            pltpu.SemaphoreType.DMA((2,)),
            pltpu.SemaphoreType.DMA((2,)),
        ],
    )
    def k(idx_hbm, table_hbm, out_hbm, idx_v, rows_v, tr_v, gsem, osem):
        wid = lax.axis_index("s") * _NUM_CORES + lax.axis_index("c")
        pltpu.sync_copy(idx_hbm.at[wid], idx_v)
        lane_iota = lax.iota(jnp.int32, 16)

        def start_gather(l, b):
            pltpu.async_copy(table_hbm.at[idx_v.at[l]], rows_v.at[b], gsem.at[b])

        def wait_gather(b):
            pltpu.make_async_copy(
                table_hbm.at[pl.ds(0, _LANES)], rows_v.at[b], gsem.at[b]
            ).wait()

        def transpose(b):
            for e in range(embed):
                col = jnp.full((16,), e, jnp.int32)
                for g in range(8):
                    v = plsc.load_gather(rows_v.at[b], [g * 16 + lane_iota, col])
                    tr_v[b, e // 8, e % 8, pl.ds(g * 16, 16)] = v

        def start_out(l, b):
            for eb in range(e_blocks):
                pltpu.async_copy(
                    tr_v.at[b, eb], out_hbm.at[l, eb, wid], osem.at[b]
                )

        def wait_out(b):
            for eb in range(e_blocks):
                pltpu.make_async_copy(
                    tr_v.at[b, eb], out_hbm.at[0, eb, 0], osem.at[b]
                ).wait()

        def step(l, b, first, last):
            @pl.when(jnp.logical_not(last))
            def _():
                start_gather(l + 1, 1 - b)

            wait_gather(b)

            @pl.when(jnp.logical_not(first))
            def _():
                wait_out(b)

            transpose(b)
            start_out(l, b)

        start_gather(0, 0)

        def pair(p, carry):
            l0 = 2 * p
            step(l0, 0, p == 0, jnp.array(False))
            step(l0 + 1, 1, p == 0, l0 + 1 == seq - 1)
            return carry

        assert seq % 2 == 0
        lax.fori_loop(0, seq // 2, pair, 0)
        wait_out(0)
        wait_out(1)

    return k(idx, table_pad)


def kernel(token_ids, table):
    b, l = token_ids.shape
    vocab, embed = table.shape
    assert b % (_NUM_WORKERS * _LANES // 32) == 0 and embed % 8 == 0
    idx5 = (
        token_ids.astype(jnp.int32)
        .T.reshape(l, _NUM_WORKERS, _LANES)
        .transpose(1, 0, 2)
    )
    eyepad = jnp.eye(embed, _LANES, dtype=jnp.float32)
    table_pad = _pad_tc(table.T, eyepad, vocab=vocab, embed=embed)
    out5 = _gather_sc(idx5, table_pad, seq=l, embed=embed)
    return jnp.transpose(out5, (2, 4, 0, 1, 3)).reshape(b, l, embed)


# speedup vs baseline: 1.3751x; 1.3751x over previous
import functools

import jax
import jax.numpy as jnp
from jax import lax
from jax.experimental import pallas as pl
from jax.experimental.pallas import tpu as pltpu
from jax.experimental.pallas import tpu_sc as plsc

_NUM_CORES = 2
_NUM_SUBCORES = 16
_NUM_WORKERS = _NUM_CORES * _NUM_SUBCORES
_CHUNK = 256
_NBUF = 3
_LANES = 128
_BLK = 16384


def _pad_body(t_ref, o_ref):
    xt = t_ref[...].T
    o_ref[...] = jnp.concatenate(
        [xt, jnp.zeros((xt.shape[0], _LANES - xt.shape[1]), jnp.float32)], axis=1
    )


@functools.partial(jax.jit, static_argnames=("vocab", "embed"))
def _pad_tc(table_t, *, vocab, embed):
    return pl.pallas_call(
        _pad_body,
        grid=(pl.cdiv(vocab, _BLK),),
        in_specs=[pl.BlockSpec((embed, _BLK), lambda i: (0, i))],
        out_specs=pl.BlockSpec((_BLK, _LANES), lambda i: (i, 0)),
        out_shape=jax.ShapeDtypeStruct((vocab, _LANES), jnp.float32),
    )(table_t)


@functools.partial(jax.jit, static_argnames=("n_chunks", "embed"))
def _gather_sc(idx, table_pad, *, n_chunks, embed):
    mesh = plsc.VectorSubcoreMesh(core_axis_name="c", subcore_axis_name="s")

    @functools.partial(
        pl.kernel,
        out_type=jax.ShapeDtypeStruct(
            (_NUM_WORKERS, n_chunks, _CHUNK, embed), jnp.float32
        ),
        mesh=mesh,
        compiler_params=pltpu.CompilerParams(use_tc_tiling_on_sc=False),
        scratch_types=[
            pltpu.VMEM((n_chunks, _CHUNK), jnp.int32),
            pltpu.VMEM((_NBUF, _CHUNK, _LANES), jnp.float32),
            pltpu.SemaphoreType.DMA((_NBUF,)),
            pltpu.SemaphoreType.DMA((_NBUF,)),
        ],
    )
    def k(idx_hbm, table_hbm, out_hbm, idx_v, rows_v, gsem, osem):
        wid = lax.axis_index("s") * _NUM_CORES + lax.axis_index("c")
        pltpu.sync_copy(idx_hbm.at[wid], idx_v)

        gathers = [None] * n_chunks
        outs = [None] * n_chunks

        def start_out(g):
            b = g % _NBUF
            return pltpu.async_copy(
                rows_v.at[b, slice(None), pl.ds(0, embed)],
                out_hbm.at[wid, g],
                osem.at[b],
            )

        for g in range(n_chunks):
            b = g % _NBUF
            if g >= _NBUF:
                outs[g - _NBUF].wait()
            gathers[g] = pltpu.async_copy(
                table_hbm.at[idx_v.at[g]], rows_v.at[b], gsem.at[b]
            )
            if g >= 1:
                gathers[g - 1].wait()
                outs[g - 1] = start_out(g - 1)
        gathers[n_chunks - 1].wait()
        outs[n_chunks - 1] = start_out(n_chunks - 1)
        for g in range(max(0, n_chunks - _NBUF), n_chunks):
            outs[g].wait()

    return k(idx, table_pad)


def kernel(token_ids, table):
    b, l = token_ids.shape
    vocab, embed = table.shape
    n = b * l
    assert n % (_NUM_WORKERS * _CHUNK) == 0
    n_chunks = n // (_NUM_WORKERS * _CHUNK)
    idx = token_ids.astype(jnp.int32).reshape(_NUM_WORKERS, n_chunks, _CHUNK)
    table_pad = _pad_tc(table.T, vocab=vocab, embed=embed)
    out = _gather_sc(idx, table_pad, n_chunks=n_chunks, embed=embed)
    return out.reshape(b, l, embed)
